# Initial kernel scaffold; baseline (speedup 1.0000x reference)
import jax
import jax.numpy as jnp
from jax.experimental import pallas as pl


def kernel(x, edge_index, gi1_W, gi1_as, gi1_ad, gi1_b, gi2_W, gi2_as, gi2_ad, gi2_b, go1_W, go1_as, go1_ad, go1_b, go2_W, go2_as, go2_ad, go2_b, ae_e1_W, ae_e1_b, ae_bn1_g, ae_bn1_b, ae_e2_W, ae_e2_b, ae_bn2_g, ae_bn2_b, ae_d1_W, ae_d1_b, ae_d2_W, ae_d2_b):
    raise NotImplementedError("write your pallas kernel here")


# speedup vs baseline: 22.0724x; 22.0724x over previous
import functools

import jax
import jax.numpy as jnp
from jax import lax
from jax.experimental import pallas as pl
from jax.experimental.pallas import tpu as pltpu
from jax.experimental.pallas import tpu_sc as plsc

_pallas_call = pl.pallas_call
_sc_kernel = pl.kernel

H = 32
NC = 2
NS = 16
NW = NC * NS
C1 = 128
C2 = 64
NBLK = 1024


def _leaky(t):
    return jnp.maximum(t, 0.2 * t)


def _elu(t):
    return jnp.where(t > 0, t, jnp.exp(jnp.minimum(t, 0.0)) - 1.0)


def _pack_uv(es_a, ed_a, es_b, ed_b):
    z = jnp.zeros_like(es_a)
    m_a = _leaky(jnp.max(es_a, axis=0, keepdims=True) + ed_a)
    m_b = _leaky(jnp.max(es_b, axis=0, keepdims=True) + ed_b)
    u = jnp.concatenate([es_a, ed_b, m_b, z], axis=1)
    v = jnp.concatenate([ed_a, m_a, es_b, z], axis=1)
    return u, v


def _stage1_body(x_ref, wgi, asgi, adgi, wgo, asgo, adgo,
                 ae1w, ae1b, bn1g, bn1b, ae2w, ae2b, bn2g, bn2b,
                 d1w, d1b, d2w, d2b,
                 hgi_o, esgi_o, edgi_o, hgo_o, esgo_o, edgo_o,
                 e1_o, e2_o, zre_o):
    x = x_ref[...]
    hgi = jnp.dot(x, wgi[...], preferred_element_type=jnp.float32)
    hgi_o[...] = hgi
    esgi_o[...] = jnp.dot(hgi, asgi[...], preferred_element_type=jnp.float32)
    edgi_o[...] = jnp.dot(hgi, adgi[...], preferred_element_type=jnp.float32)
    hgo = jnp.dot(x, wgo[...], preferred_element_type=jnp.float32)
    hgo_o[...] = hgo
    esgo_o[...] = jnp.dot(hgo, asgo[...], preferred_element_type=jnp.float32)
    edgo_o[...] = jnp.dot(hgo, adgo[...], preferred_element_type=jnp.float32)
    bn_scale = 1.0 / jnp.sqrt(jnp.float32(1.0 + 1e-5))
    e1 = _elu(jnp.dot(x, ae1w[...], preferred_element_type=jnp.float32)
              + ae1b[...])
    e1 = e1 * bn_scale * bn1g[...] + bn1b[...]
    e1_o[...] = e1
    e2 = _elu(jnp.dot(e1, ae2w[...], preferred_element_type=jnp.float32)
              + ae2b[...])
    e2 = e2 * bn_scale * bn2g[...] + bn2b[...]
    e2_o[...] = e2
    d1 = _elu(jnp.dot(e2, d1w[...], preferred_element_type=jnp.float32)
              + d1b[...])
    zre_o[...] = jax.nn.sigmoid(
        jnp.dot(d1, d2w[...], preferred_element_type=jnp.float32) + d2b[...])


def _stage1(xp, wgi, asgi, adgi, wgo, asgo, adgo, ae):
    np_, din = xp.shape
    hw = wgi.shape[1]
    grid = np_ // NBLK

    def wspec(a):
        return pl.BlockSpec(a.shape, lambda i: tuple(0 for _ in a.shape))

    args = (wgi, asgi, adgi, wgo, asgo, adgo) + ae
    out_shapes = [
        jax.ShapeDtypeStruct((np_, hw), jnp.float32),
        jax.ShapeDtypeStruct((np_, H), jnp.float32),
        jax.ShapeDtypeStruct((np_, H), jnp.float32),
        jax.ShapeDtypeStruct((np_, hw), jnp.float32),
        jax.ShapeDtypeStruct((np_, H), jnp.float32),
        jax.ShapeDtypeStruct((np_, H), jnp.float32),
        jax.ShapeDtypeStruct((np_, 16), jnp.float32),
        jax.ShapeDtypeStruct((np_, 8), jnp.float32),
        jax.ShapeDtypeStruct((np_, din), jnp.float32),
    ]
    return _pallas_call(
        _stage1_body,
        grid=(grid,),
        in_specs=[pl.BlockSpec((NBLK, din), lambda i: (i, 0))]
                 + [wspec(a) for a in args],
        out_specs=[pl.BlockSpec((NBLK, s.shape[1]), lambda i: (i, 0))
                   for s in out_shapes],
        out_shape=out_shapes,
    )(xp, *args)


def _tables_body(esgi, edgi, esgo, edgo, u_o, v_o):
    u, v = _pack_uv(esgi[...], edgi[...], esgo[...], edgo[...])
    u_o[...] = u
    v_o[...] = v


def _tables(esgi, edgi, esgo, edgo):
    np_ = esgi.shape[0]
    out = [jax.ShapeDtypeStruct((np_, 4 * H), jnp.float32)] * 2
    return _pallas_call(_tables_body, out_shape=out)(esgi, edgi, esgo, edgo)


def _ivd_body(dena, denb, w_o):
    ivda = 1.0 / (dena[0] + dena[1] + 1e-16)
    ivdb = 1.0 / (denb[0] + denb[1] + 1e-16)
    w_o[...] = jnp.concatenate(
        [ivda, ivdb, jnp.zeros_like(ivda), jnp.zeros_like(ivda)], axis=1)


def _ivd(dena, denb):
    np_ = dena.shape[1]
    out = jax.ShapeDtypeStruct((np_, 4 * H), jnp.float32)
    return _pallas_call(_ivd_body, out_shape=out)(dena, denb)


def _layer2_body(numgi, numgo, bgi, bgo, wgi2, asgi2, adgi2, wgo2, asgo2,
                 adgo2, z1_o, o1_o, hgi2_o, esgi2_o, edgi2_o, hgo2_o,
                 esgo2_o, edgo2_o):
    z1 = _elu((numgi[0] + numgi[1]) * (1.0 / H) + bgi[...])
    o1 = _elu((numgo[0] + numgo[1]) * (1.0 / H) + bgo[...])
    z1_o[...] = z1
    o1_o[...] = o1
    hgi2 = jnp.dot(z1, wgi2[...], preferred_element_type=jnp.float32)
    hgi2_o[...] = hgi2
    esgi2_o[...] = jnp.dot(hgi2, asgi2[...], preferred_element_type=jnp.float32)
    edgi2_o[...] = jnp.dot(hgi2, adgi2[...], preferred_element_type=jnp.float32)
    hgo2 = jnp.dot(o1, wgo2[...], preferred_element_type=jnp.float32)
    hgo2_o[...] = hgo2
    esgo2_o[...] = jnp.dot(hgo2, asgo2[...], preferred_element_type=jnp.float32)
    edgo2_o[...] = jnp.dot(hgo2, adgo2[...], preferred_element_type=jnp.float32)


def _layer2(numgi, numgo, bgi, bgo, wgi2, asgi2, adgi2, wgo2, asgo2, adgo2):
    np_ = numgi.shape[1]
    hw2 = wgi2.shape[1]
    ch = wgi2.shape[0]
    grid = np_ // NBLK

    def wspec(a):
        return pl.BlockSpec(a.shape, lambda i: tuple(0 for _ in a.shape))

    num_spec = pl.BlockSpec((NC, NBLK, ch), lambda i: (0, i, 0))
    wargs = (bgi, bgo, wgi2, asgi2, adgi2, wgo2, asgo2, adgo2)
    out_shapes = [
        jax.ShapeDtypeStruct((np_, ch), jnp.float32),
        jax.ShapeDtypeStruct((np_, ch), jnp.float32),
        jax.ShapeDtypeStruct((np_, hw2), jnp.float32),
        jax.ShapeDtypeStruct((np_, H), jnp.float32),
        jax.ShapeDtypeStruct((np_, H), jnp.float32),
        jax.ShapeDtypeStruct((np_, hw2), jnp.float32),
        jax.ShapeDtypeStruct((np_, H), jnp.float32),
        jax.ShapeDtypeStruct((np_, H), jnp.float32),
    ]
    return _pallas_call(
        _layer2_body,
        grid=(grid,),
        in_specs=[num_spec, num_spec] + [wspec(a) for a in wargs],
        out_specs=[pl.BlockSpec((NBLK, s.shape[1]), lambda i: (i, 0))
                   for s in out_shapes],
        out_shape=out_shapes,
    )(numgi, numgo, *wargs)


def _final_body(numgi2, numgo2, bgi2, bgo2, z2_o, o2_o):
    z2_o[...] = _elu((numgi2[0] + numgi2[1]) * (1.0 / H) + bgi2[...])
    o2_o[...] = _elu((numgo2[0] + numgo2[1]) * (1.0 / H) + bgo2[...])


def _final(numgi2, numgo2, bgi2, bgo2):
    out = [jax.ShapeDtypeStruct(numgi2.shape[1:], jnp.float32)] * 2
    return _pallas_call(_final_body, out_shape=out)(
        numgi2, numgo2, bgi2, bgo2)


def _scatter_add_rows(src_vmem, shared_ref, idx_ref):
    pltpu.sync_copy(src_vmem, shared_ref.at[idx_ref], add=True)


def _sc_mesh():
    return plsc.VectorSubcoreMesh(core_axis_name="c", subcore_axis_name="s")


def _pass1(s_ids, d_ids, u_t, v_t, zeros32):
    ep = s_ids.shape[0]
    np_ = u_t.shape[0]
    ew = ep // NW
    nchunk = ew // C1
    rpt = np_ // NS

    @functools.partial(
        _sc_kernel, mesh=_sc_mesh(),
        compiler_params=pltpu.CompilerParams(use_tc_tiling_on_sc=False,
                                             needs_layout_passes=False),
        out_type=[jax.ShapeDtypeStruct((ep, H), jnp.float32),
                  jax.ShapeDtypeStruct((ep, H), jnp.float32),
                  jax.ShapeDtypeStruct((NC, np_, H), jnp.float32),
                  jax.ShapeDtypeStruct((NC, np_, H), jnp.float32)],
        scratch_types=[
            pltpu.VMEM((C1,), jnp.int32), pltpu.VMEM((C1,), jnp.int32),
            pltpu.VMEM((C1, 4 * H), jnp.float32),
            pltpu.VMEM((C1, 4 * H), jnp.float32),
            pltpu.VMEM((C1, H), jnp.float32), pltpu.VMEM((C1, H), jnp.float32),
            pltpu.VMEM_SHARED((np_, H), jnp.float32),
            pltpu.VMEM_SHARED((np_, H), jnp.float32),
            pltpu.SemaphoreType.DMA, pltpu.SemaphoreType.DMA,
        ],
    )
    def kern(s_hbm, d_hbm, u_hbm, v_hbm, z_hbm,
             exa_hbm, exb_hbm, dena_hbm, denb_hbm,
             sidx, didx, ur, vr, exa_v, exb_v,
             dsha, dshb, sem1, sem2):
        c = lax.axis_index("c")
        s_ = lax.axis_index("s")
        wid = c * NS + s_
        pltpu.sync_copy(z_hbm.at[pl.ds(s_ * rpt, rpt)],
                        dsha.at[pl.ds(s_ * rpt, rpt)])
        pltpu.sync_copy(z_hbm.at[pl.ds(s_ * rpt, rpt)],
                        dshb.at[pl.ds(s_ * rpt, rpt)])
        plsc.subcore_barrier()

        def chunk(j, _):
            base = wid * ew + j * C1
            pltpu.sync_copy(s_hbm.at[pl.ds(base, C1)], sidx)
            pltpu.sync_copy(d_hbm.at[pl.ds(base, C1)], didx)
            ga = pltpu.async_copy(u_hbm.at[sidx], ur, sem1)
            gb = pltpu.async_copy(v_hbm.at[didx], vr, sem2)
            ga.wait()
            gb.wait()

            def row(i, _):
                for kk in range(H // 16):
                    o = 16 * kk
                    ta = ur[i, pl.ds(o, 16)] + vr[i, pl.ds(o, 16)]
                    exa_v[i, pl.ds(o, 16)] = jnp.exp(
                        jnp.maximum(ta, 0.2 * ta) - vr[i, pl.ds(H + o, 16)])
                    tb = vr[i, pl.ds(2 * H + o, 16)] + ur[i, pl.ds(H + o, 16)]
                    exb_v[i, pl.ds(o, 16)] = jnp.exp(
                        jnp.maximum(tb, 0.2 * tb) - ur[i, pl.ds(2 * H + o, 16)])
                return 0

            lax.fori_loop(0, C1, row, 0)
            pltpu.sync_copy(exa_v, exa_hbm.at[pl.ds(base, C1)])
            pltpu.sync_copy(exb_v, exb_hbm.at[pl.ds(base, C1)])
            _scatter_add_rows(exa_v, dsha, didx)
            _scatter_add_rows(exb_v, dshb, sidx)
            return 0

        lax.fori_loop(0, nchunk, chunk, 0)
        plsc.subcore_barrier()
        pltpu.sync_copy(dsha.at[pl.ds(s_ * rpt, rpt)],
                        dena_hbm.at[c, pl.ds(s_ * rpt, rpt)])
        pltpu.sync_copy(dshb.at[pl.ds(s_ * rpt, rpt)],
                        denb_hbm.at[c, pl.ds(s_ * rpt, rpt)])

    return kern(s_ids, d_ids, u_t, v_t, zeros32)


def _pass2(s_ids, d_ids, h_a, h_b, ex_a, ex_b, w_t, zeros_ch, ch):
    ep = s_ids.shape[0]
    np_ = h_a.shape[0]
    hw = h_a.shape[1]
    ew = ep // NW
    nchunk = ew // C2
    rpt = np_ // NS

    @functools.partial(
        _sc_kernel, mesh=_sc_mesh(),
        compiler_params=pltpu.CompilerParams(use_tc_tiling_on_sc=False,
                                             needs_layout_passes=False),
        out_type=[jax.ShapeDtypeStruct((NC, np_, ch), jnp.float32),
                  jax.ShapeDtypeStruct((NC, np_, ch), jnp.float32)],
        scratch_types=[
            pltpu.VMEM((C2,), jnp.int32), pltpu.VMEM((C2,), jnp.int32),
            pltpu.VMEM((C2, hw), jnp.float32), pltpu.VMEM((C2, hw), jnp.float32),
            pltpu.VMEM((C2, 4 * H), jnp.float32),
            pltpu.VMEM((C2, 4 * H), jnp.float32),
            pltpu.VMEM((C2, H), jnp.float32), pltpu.VMEM((C2, H), jnp.float32),
            pltpu.VMEM((C2, H), jnp.float32), pltpu.VMEM((C2, H), jnp.float32),
            pltpu.VMEM((C2, ch), jnp.float32), pltpu.VMEM((C2, ch), jnp.float32),
            pltpu.VMEM_SHARED((np_, ch), jnp.float32),
            pltpu.VMEM_SHARED((np_, ch), jnp.float32),
            pltpu.SemaphoreType.DMA, pltpu.SemaphoreType.DMA,
            pltpu.SemaphoreType.DMA, pltpu.SemaphoreType.DMA,
        ],
    )
    def kern(s_hbm, d_hbm, ha_hbm, hb_hbm, exa_hbm, exb_hbm, w_hbm, z_hbm,
             numa_hbm, numb_hbm,
             sidx, didx, hra, hrb, wsr, wdr, exa_v, exb_v, wa, wb,
             ma, mb, nsha, nshb, sem1, sem2, sem3, sem4):
        c = lax.axis_index("c")
        s_ = lax.axis_index("s")
        wid = c * NS + s_
        pltpu.sync_copy(z_hbm.at[pl.ds(s_ * rpt, rpt)],
                        nsha.at[pl.ds(s_ * rpt, rpt)])
        pltpu.sync_copy(z_hbm.at[pl.ds(s_ * rpt, rpt)],
                        nshb.at[pl.ds(s_ * rpt, rpt)])
        plsc.subcore_barrier()

        lmask = lax.iota(jnp.int32, 16) < 8
        pat8 = lax.iota(jnp.int32, 16) & 7

        def reduce_heads(hr, w, m):
            if ch == 16:
                def edge(i, _):
                    wv = [w[i, pl.ds(0, 16)], w[i, pl.ds(16, 16)]]
                    acc = jnp.zeros((16,), jnp.float32)
                    for hh in range(H):
                        acc = acc + (wv[hh // 16][hh % 16]
                                     * hr[i, pl.ds(16 * hh, 16)])
                    m[i, :] = acc
                    return 0
                lax.fori_loop(0, C2, edge, 0)
            else:
                def pair(i2, _):
                    r0 = 2 * i2
                    rowidx = jnp.where(lmask, r0, r0 + 1)
                    wlo = [w[r0, pl.ds(0, 16)], w[r0, pl.ds(16, 16)]]
                    whi = [w[r0 + 1, pl.ds(0, 16)], w[r0 + 1, pl.ds(16, 16)]]
                    acc = jnp.zeros((16,), jnp.float32)
                    for hh in range(H):
                        hv = plsc.load_gather(hr, [rowidx, pat8 + 8 * hh])
                        wv = jnp.where(lmask, wlo[hh // 16][hh % 16],
                                       whi[hh // 16][hh % 16])
                        acc = acc + wv * hv
                    plsc.store_scatter(m, [rowidx, pat8], acc)
                    return 0
                lax.fori_loop(0, C2 // 2, pair, 0)

        def chunk(j, _):
            base = wid * ew + j * C2
            pltpu.sync_copy(s_hbm.at[pl.ds(base, C2)], sidx)
            pltpu.sync_copy(d_hbm.at[pl.ds(base, C2)], didx)
            ga = pltpu.async_copy(ha_hbm.at[sidx], hra, sem1)
            gb = pltpu.async_copy(w_hbm.at[didx], wdr, sem2)
            gc = pltpu.async_copy(hb_hbm.at[didx], hrb, sem3)
            gd = pltpu.async_copy(w_hbm.at[sidx], wsr, sem4)
            pltpu.sync_copy(exa_hbm.at[pl.ds(base, C2)], exa_v)
            pltpu.sync_copy(exb_hbm.at[pl.ds(base, C2)], exb_v)
            ga.wait()
            gb.wait()
            gc.wait()
            gd.wait()

            def wrow(i, _):
                for kk in range(H // 16):
                    o = 16 * kk
                    wa[i, pl.ds(o, 16)] = (exa_v[i, pl.ds(o, 16)]
                                           * wdr[i, pl.ds(o, 16)])
                    wb[i, pl.ds(o, 16)] = (exb_v[i, pl.ds(o, 16)]
                                           * wsr[i, pl.ds(H + o, 16)])
                return 0

            lax.fori_loop(0, C2, wrow, 0)
            reduce_heads(hra, wa, ma)
            reduce_heads(hrb, wb, mb)
            _scatter_add_rows(ma, nsha, didx)
            _scatter_add_rows(mb, nshb, sidx)
            return 0

        lax.fori_loop(0, nchunk, chunk, 0)
        plsc.subcore_barrier()
        pltpu.sync_copy(nsha.at[pl.ds(s_ * rpt, rpt)],
                        numa_hbm.at[c, pl.ds(s_ * rpt, rpt)])
        pltpu.sync_copy(nshb.at[pl.ds(s_ * rpt, rpt)],
                        numb_hbm.at[c, pl.ds(s_ * rpt, rpt)])

    return kern(s_ids, d_ids, h_a, h_b, ex_a, ex_b, w_t, zeros_ch)


def _blockdiag(a):
    ch = a.shape[1]
    eye = jnp.eye(H, dtype=a.dtype)
    return (a[:, :, None] * eye[:, None, :]).reshape(H * ch, H)


def kernel(x, edge_index, gi1_W, gi1_as, gi1_ad, gi1_b, gi2_W, gi2_as,
           gi2_ad, gi2_b, go1_W, go1_as, go1_ad, go1_b, go2_W, go2_as,
           go2_ad, go2_b, ae_e1_W, ae_e1_b, ae_bn1_g, ae_bn1_b, ae_e2_W,
           ae_e2_b, ae_bn2_g, ae_bn2_b, ae_d1_W, ae_d1_b, ae_d2_W, ae_d2_b):
    n, din = x.shape
    e = edge_index.shape[1]
    np_ = -(-(n + 1) // NBLK) * NBLK
    ep = -(-e // (NW * C1)) * (NW * C1)

    xp = jnp.pad(x, ((0, np_ - n), (0, 0)))
    pad_ids = jnp.full((ep - e,), n, dtype=jnp.int32)
    s_ids = jnp.concatenate([edge_index[0], pad_ids])
    d_ids = jnp.concatenate([edge_index[1], pad_ids])
    zeros32 = jnp.zeros((np_, H), jnp.float32)
    zeros16 = jnp.zeros((np_, 16), jnp.float32)
    zeros8 = jnp.zeros((np_, 8), jnp.float32)

    ae = (ae_e1_W.T, ae_e1_b[None, :], ae_bn1_g[None, :], ae_bn1_b[None, :],
          ae_e2_W.T, ae_e2_b[None, :], ae_bn2_g[None, :], ae_bn2_b[None, :],
          ae_d1_W.T, ae_d1_b[None, :], ae_d2_W.T, ae_d2_b[None, :])

    (hgi, esgi, edgi, hgo, esgo, edgo, e1, e2, zre) = _stage1(
        xp, gi1_W.T, _blockdiag(gi1_as), _blockdiag(gi1_ad),
        go1_W.T, _blockdiag(go1_as), _blockdiag(go1_ad), ae)

    u1, v1 = _tables(esgi, edgi, esgo, edgo)

    exgi, exgo, dengi, dengo = _pass1(s_ids, d_ids, u1, v1, zeros32)
    w1 = _ivd(dengi, dengo)
    numgi, numgo = _pass2(s_ids, d_ids, hgi, hgo, exgi, exgo, w1, zeros16, 16)

    (z1, o1, hgi2, esgi2, edgi2, hgo2, esgo2, edgo2) = _layer2(
        numgi, numgo, gi1_b[None, :], go1_b[None, :],
        gi2_W.T, _blockdiag(gi2_as), _blockdiag(gi2_ad),
        go2_W.T, _blockdiag(go2_as), _blockdiag(go2_ad))
    u2, v2 = _tables(esgi2, edgi2, esgo2, edgo2)

    exgi2, exgo2, dengi2, dengo2 = _pass1(s_ids, d_ids, u2, v2, zeros32)
    w2 = _ivd(dengi2, dengo2)
    numgi2, numgo2 = _pass2(s_ids, d_ids, hgi2, hgo2, exgi2, exgo2, w2,
                            zeros8, 8)

    z2, o2 = _final(numgi2, numgo2, gi2_b[None, :], go2_b[None, :])

    x_in = jnp.concatenate([z1[:n], z2[:n]], axis=-1)
    x_out = jnp.concatenate([o1[:n], o2[:n]], axis=-1)
    x_self = jnp.concatenate([e1[:n], e2[:n]], axis=-1)
    z_self_re = zre[:n]
    return (x_in, x_out, x_self, z_self_re)


# speedup vs baseline: 35.6454x; 1.6149x over previous
import functools

import jax
import jax.numpy as jnp
from jax import lax
from jax.experimental import pallas as pl
from jax.experimental.pallas import tpu as pltpu
from jax.experimental.pallas import tpu_sc as plsc

_pallas_call = pl.pallas_call
_sc_kernel = pl.kernel

H = 32
NC = 2
NS = 16
NW = NC * NS
C1 = 128
C2 = 64
NBLK = 1024


def _leaky(t):
    return jnp.maximum(t, 0.2 * t)


def _elu(t):
    return jnp.where(t > 0, t, jnp.exp(jnp.minimum(t, 0.0)) - 1.0)


def _pack_uv(es_a, ed_a, es_b, ed_b):
    z = jnp.zeros_like(es_a)
    m_a = _leaky(jnp.max(es_a, axis=0, keepdims=True) + ed_a)
    m_b = _leaky(jnp.max(es_b, axis=0, keepdims=True) + ed_b)
    u = jnp.concatenate([es_a, ed_b, m_b, z], axis=1)
    v = jnp.concatenate([ed_a, m_a, es_b, z], axis=1)
    return u, v


def _stage1_body(x_ref, wgi, asgi, adgi, wgo, asgo, adgo,
                 ae1w, ae1b, bn1g, bn1b, ae2w, ae2b, bn2g, bn2b,
                 d1w, d1b, d2w, d2b,
                 hgi_o, esgi_o, edgi_o, hgo_o, esgo_o, edgo_o,
                 e1_o, e2_o, zre_o):
    x = x_ref[...]
    hgi = jnp.dot(x, wgi[...], preferred_element_type=jnp.float32)
    hgi_o[...] = hgi
    esgi_o[...] = jnp.dot(hgi, asgi[...], preferred_element_type=jnp.float32)
    edgi_o[...] = jnp.dot(hgi, adgi[...], preferred_element_type=jnp.float32)
    hgo = jnp.dot(x, wgo[...], preferred_element_type=jnp.float32)
    hgo_o[...] = hgo
    esgo_o[...] = jnp.dot(hgo, asgo[...], preferred_element_type=jnp.float32)
    edgo_o[...] = jnp.dot(hgo, adgo[...], preferred_element_type=jnp.float32)
    bn_scale = 1.0 / jnp.sqrt(jnp.float32(1.0 + 1e-5))
    e1 = _elu(jnp.dot(x, ae1w[...], preferred_element_type=jnp.float32)
              + ae1b[...])
    e1 = e1 * bn_scale * bn1g[...] + bn1b[...]
    e1_o[...] = e1
    e2 = _elu(jnp.dot(e1, ae2w[...], preferred_element_type=jnp.float32)
              + ae2b[...])
    e2 = e2 * bn_scale * bn2g[...] + bn2b[...]
    e2_o[...] = e2
    d1 = _elu(jnp.dot(e2, d1w[...], preferred_element_type=jnp.float32)
              + d1b[...])
    zre_o[...] = jax.nn.sigmoid(
        jnp.dot(d1, d2w[...], preferred_element_type=jnp.float32) + d2b[...])


def _stage1(xp, wgi, asgi, adgi, wgo, asgo, adgo, ae):
    np_, din = xp.shape
    hw = wgi.shape[1]
    grid = np_ // NBLK

    def wspec(a):
        return pl.BlockSpec(a.shape, lambda i: tuple(0 for _ in a.shape))

    args = (wgi, asgi, adgi, wgo, asgo, adgo) + ae
    out_shapes = [
        jax.ShapeDtypeStruct((np_, hw), jnp.float32),
        jax.ShapeDtypeStruct((np_, H), jnp.float32),
        jax.ShapeDtypeStruct((np_, H), jnp.float32),
        jax.ShapeDtypeStruct((np_, hw), jnp.float32),
        jax.ShapeDtypeStruct((np_, H), jnp.float32),
        jax.ShapeDtypeStruct((np_, H), jnp.float32),
        jax.ShapeDtypeStruct((np_, 16), jnp.float32),
        jax.ShapeDtypeStruct((np_, 8), jnp.float32),
        jax.ShapeDtypeStruct((np_, din), jnp.float32),
    ]
    return _pallas_call(
        _stage1_body,
        grid=(grid,),
        in_specs=[pl.BlockSpec((NBLK, din), lambda i: (i, 0))]
                 + [wspec(a) for a in args],
        out_specs=[pl.BlockSpec((NBLK, s.shape[1]), lambda i: (i, 0))
                   for s in out_shapes],
        out_shape=out_shapes,
    )(xp, *args)


def _tables_body(esgi, edgi, esgo, edgo, u_o, v_o):
    u, v = _pack_uv(esgi[...], edgi[...], esgo[...], edgo[...])
    u_o[...] = u
    v_o[...] = v


def _tables(esgi, edgi, esgo, edgo):
    np_ = esgi.shape[0]
    out = [jax.ShapeDtypeStruct((np_, 4 * H), jnp.float32)] * 2
    return _pallas_call(_tables_body, out_shape=out)(esgi, edgi, esgo, edgo)


def _ivd_body(dena, denb, w_o):
    ivda = 1.0 / (dena[0] + dena[1] + 1e-16)
    ivdb = 1.0 / (denb[0] + denb[1] + 1e-16)
    w_o[...] = jnp.concatenate(
        [ivda, ivdb, jnp.zeros_like(ivda), jnp.zeros_like(ivda)], axis=1)


def _ivd(dena, denb):
    np_ = dena.shape[1]
    out = jax.ShapeDtypeStruct((np_, 4 * H), jnp.float32)
    return _pallas_call(_ivd_body, out_shape=out)(dena, denb)


def _layer2_body(numgi, numgo, bgi, bgo, wgi2, asgi2, adgi2, wgo2, asgo2,
                 adgo2, z1_o, o1_o, hgi2_o, esgi2_o, edgi2_o, hgo2_o,
                 esgo2_o, edgo2_o):
    z1 = _elu((numgi[0] + numgi[1]) * (1.0 / H) + bgi[...])
    o1 = _elu((numgo[0] + numgo[1]) * (1.0 / H) + bgo[...])
    z1_o[...] = z1
    o1_o[...] = o1
    hgi2 = jnp.dot(z1, wgi2[...], preferred_element_type=jnp.float32)
    hgi2_o[...] = hgi2
    esgi2_o[...] = jnp.dot(hgi2, asgi2[...], preferred_element_type=jnp.float32)
    edgi2_o[...] = jnp.dot(hgi2, adgi2[...], preferred_element_type=jnp.float32)
    hgo2 = jnp.dot(o1, wgo2[...], preferred_element_type=jnp.float32)
    hgo2_o[...] = hgo2
    esgo2_o[...] = jnp.dot(hgo2, asgo2[...], preferred_element_type=jnp.float32)
    edgo2_o[...] = jnp.dot(hgo2, adgo2[...], preferred_element_type=jnp.float32)


def _layer2(numgi, numgo, bgi, bgo, wgi2, asgi2, adgi2, wgo2, asgo2, adgo2):
    np_ = numgi.shape[1]
    hw2 = wgi2.shape[1]
    ch = wgi2.shape[0]
    grid = np_ // NBLK

    def wspec(a):
        return pl.BlockSpec(a.shape, lambda i: tuple(0 for _ in a.shape))

    num_spec = pl.BlockSpec((NC, NBLK, ch), lambda i: (0, i, 0))
    wargs = (bgi, bgo, wgi2, asgi2, adgi2, wgo2, asgo2, adgo2)
    out_shapes = [
        jax.ShapeDtypeStruct((np_, ch), jnp.float32),
        jax.ShapeDtypeStruct((np_, ch), jnp.float32),
        jax.ShapeDtypeStruct((np_, hw2), jnp.float32),
        jax.ShapeDtypeStruct((np_, H), jnp.float32),
        jax.ShapeDtypeStruct((np_, H), jnp.float32),
        jax.ShapeDtypeStruct((np_, hw2), jnp.float32),
        jax.ShapeDtypeStruct((np_, H), jnp.float32),
        jax.ShapeDtypeStruct((np_, H), jnp.float32),
    ]
    return _pallas_call(
        _layer2_body,
        grid=(grid,),
        in_specs=[num_spec, num_spec] + [wspec(a) for a in wargs],
        out_specs=[pl.BlockSpec((NBLK, s.shape[1]), lambda i: (i, 0))
                   for s in out_shapes],
        out_shape=out_shapes,
    )(numgi, numgo, *wargs)


def _final_body(numgi2, numgo2, bgi2, bgo2, z2_o, o2_o):
    z2_o[...] = _elu((numgi2[0] + numgi2[1]) * (1.0 / H) + bgi2[...])
    o2_o[...] = _elu((numgo2[0] + numgo2[1]) * (1.0 / H) + bgo2[...])


def _final(numgi2, numgo2, bgi2, bgo2):
    out = [jax.ShapeDtypeStruct(numgi2.shape[1:], jnp.float32)] * 2
    return _pallas_call(_final_body, out_shape=out)(
        numgi2, numgo2, bgi2, bgo2)


def _scatter_add_rows(src_vmem, shared_ref, idx_ref):
    pltpu.sync_copy(src_vmem, shared_ref.at[idx_ref], add=True)


def _sc_mesh():
    return plsc.VectorSubcoreMesh(core_axis_name="c", subcore_axis_name="s")


def _pass1(s_ids, d_ids, u_t, v_t, zeros32):
    ep = s_ids.shape[0]
    np_ = u_t.shape[0]
    ew = ep // NW
    nchunk = ew // C1
    rpt = np_ // NS
    s2d = s_ids.reshape(ep // C1, C1)
    d2d = d_ids.reshape(ep // C1, C1)

    @functools.partial(
        _sc_kernel, mesh=_sc_mesh(),
        compiler_params=pltpu.CompilerParams(use_tc_tiling_on_sc=False,
                                             needs_layout_passes=False),
        out_type=[jax.ShapeDtypeStruct((ep, H), jnp.float32),
                  jax.ShapeDtypeStruct((ep, H), jnp.float32),
                  jax.ShapeDtypeStruct((NC, np_, H), jnp.float32),
                  jax.ShapeDtypeStruct((NC, np_, H), jnp.float32)],
        scratch_types=[
            pltpu.VMEM((nchunk, C1), jnp.int32),
            pltpu.VMEM((nchunk, C1), jnp.int32),
            pltpu.VMEM((2, C1, 4 * H), jnp.float32),
            pltpu.VMEM((2, C1, 4 * H), jnp.float32),
            pltpu.VMEM((C1, H), jnp.float32), pltpu.VMEM((C1, H), jnp.float32),
            pltpu.VMEM_SHARED((np_, H), jnp.float32),
            pltpu.VMEM_SHARED((np_, H), jnp.float32),
            pltpu.SemaphoreType.DMA, pltpu.SemaphoreType.DMA,
        ],
    )
    def kern(s_hbm, d_hbm, u_hbm, v_hbm, z_hbm,
             exa_hbm, exb_hbm, dena_hbm, denb_hbm,
             sidx_all, didx_all, ur2, vr2, exa_v, exb_v,
             dsha, dshb, sem_a, sem_b):
        c = lax.axis_index("c")
        s_ = lax.axis_index("s")
        wid = c * NS + s_
        pltpu.sync_copy(z_hbm.at[pl.ds(s_ * rpt, rpt)],
                        dsha.at[pl.ds(s_ * rpt, rpt)])
        pltpu.sync_copy(z_hbm.at[pl.ds(s_ * rpt, rpt)],
                        dshb.at[pl.ds(s_ * rpt, rpt)])
        pltpu.sync_copy(s_hbm.at[pl.ds(wid * nchunk, nchunk)], sidx_all)
        pltpu.sync_copy(d_hbm.at[pl.ds(wid * nchunk, nchunk)], didx_all)
        plsc.subcore_barrier()

        sems = (sem_a, sem_b)

        def issue(k, b):
            pltpu.async_copy(u_hbm.at[sidx_all.at[k]], ur2.at[b], sems[b])
            pltpu.async_copy(v_hbm.at[didx_all.at[k]], vr2.at[b], sems[b])

        def drain(k, b):
            pltpu.make_async_copy(u_hbm.at[sidx_all.at[k]], ur2.at[b],
                                  sems[b]).wait()
            pltpu.make_async_copy(v_hbm.at[didx_all.at[k]], vr2.at[b],
                                  sems[b]).wait()

        def compute(j, b):
            ur = ur2.at[b]
            vr = vr2.at[b]

            def row(i, _):
                for kk in range(H // 16):
                    o = 16 * kk
                    ta = ur[i, pl.ds(o, 16)] + vr[i, pl.ds(o, 16)]
                    exa_v[i, pl.ds(o, 16)] = jnp.exp(
                        jnp.maximum(ta, 0.2 * ta) - vr[i, pl.ds(H + o, 16)])
                    tb = vr[i, pl.ds(2 * H + o, 16)] + ur[i, pl.ds(H + o, 16)]
                    exb_v[i, pl.ds(o, 16)] = jnp.exp(
                        jnp.maximum(tb, 0.2 * tb) - ur[i, pl.ds(2 * H + o, 16)])
                return 0

            lax.fori_loop(0, C1, row, 0)
            base = wid * ew + j * C1
            pltpu.sync_copy(exa_v, exa_hbm.at[pl.ds(base, C1)])
            pltpu.sync_copy(exb_v, exb_hbm.at[pl.ds(base, C1)])
            _scatter_add_rows(exa_v, dsha, didx_all.at[j])
            _scatter_add_rows(exb_v, dshb, sidx_all.at[j])

        issue(0, 0)

        def pair(j2, _):
            j = 2 * j2
            issue(j + 1, 1)
            drain(j, 0)
            compute(j, 0)

            @pl.when(j + 2 < nchunk)
            def _():
                issue(j + 2, 0)

            drain(j + 1, 1)
            compute(j + 1, 1)
            return 0

        lax.fori_loop(0, nchunk // 2, pair, 0)
        plsc.subcore_barrier()
        pltpu.sync_copy(dsha.at[pl.ds(s_ * rpt, rpt)],
                        dena_hbm.at[c, pl.ds(s_ * rpt, rpt)])
        pltpu.sync_copy(dshb.at[pl.ds(s_ * rpt, rpt)],
                        denb_hbm.at[c, pl.ds(s_ * rpt, rpt)])

    return kern(s2d, d2d, u_t, v_t, zeros32)


def _pass2(s_ids, d_ids, h_a, h_b, ex_a, ex_b, w_t, zeros_ch, ch):
    ep = s_ids.shape[0]
    np_ = h_a.shape[0]
    hw = h_a.shape[1]
    c2 = 32
    ew = ep // NW
    nchunk = ew // c2
    rpt = np_ // NS
    s2d = s_ids.reshape(ep // c2, c2)
    d2d = d_ids.reshape(ep // c2, c2)

    @functools.partial(
        _sc_kernel, mesh=_sc_mesh(),
        compiler_params=pltpu.CompilerParams(use_tc_tiling_on_sc=False,
                                             needs_layout_passes=False),
        out_type=[jax.ShapeDtypeStruct((NC, np_, ch), jnp.float32),
                  jax.ShapeDtypeStruct((NC, np_, ch), jnp.float32)],
        scratch_types=[
            pltpu.VMEM((nchunk, c2), jnp.int32),
            pltpu.VMEM((nchunk, c2), jnp.int32),
            pltpu.VMEM((2, c2, hw), jnp.float32),
            pltpu.VMEM((2, c2, hw), jnp.float32),
            pltpu.VMEM((2, c2, 4 * H), jnp.float32),
            pltpu.VMEM((2, c2, 4 * H), jnp.float32),
            pltpu.VMEM((2, c2, H), jnp.float32),
            pltpu.VMEM((2, c2, H), jnp.float32),
            pltpu.VMEM((c2, H), jnp.float32), pltpu.VMEM((c2, H), jnp.float32),
            pltpu.VMEM((c2, ch), jnp.float32), pltpu.VMEM((c2, ch), jnp.float32),
            pltpu.VMEM_SHARED((np_, ch), jnp.float32),
            pltpu.VMEM_SHARED((np_, ch), jnp.float32),
            pltpu.SemaphoreType.DMA, pltpu.SemaphoreType.DMA,
        ],
    )
    def kern(s_hbm, d_hbm, ha_hbm, hb_hbm, exa_hbm, exb_hbm, w_hbm, z_hbm,
             numa_hbm, numb_hbm,
             sidx_all, didx_all, hra2, hrb2, wsr2, wdr2, exa2, exb2, wa, wb,
             ma, mb, nsha, nshb, sem_a, sem_b):
        c = lax.axis_index("c")
        s_ = lax.axis_index("s")
        wid = c * NS + s_
        pltpu.sync_copy(z_hbm.at[pl.ds(s_ * rpt, rpt)],
                        nsha.at[pl.ds(s_ * rpt, rpt)])
        pltpu.sync_copy(z_hbm.at[pl.ds(s_ * rpt, rpt)],
                        nshb.at[pl.ds(s_ * rpt, rpt)])
        pltpu.sync_copy(s_hbm.at[pl.ds(wid * nchunk, nchunk)], sidx_all)
        pltpu.sync_copy(d_hbm.at[pl.ds(wid * nchunk, nchunk)], didx_all)
        plsc.subcore_barrier()

        lmask = lax.iota(jnp.int32, 16) < 8
        pat8 = lax.iota(jnp.int32, 16) & 7
        sems = (sem_a, sem_b)

        def issue(k, b):
            base = wid * ew + k * c2
            pltpu.async_copy(ha_hbm.at[sidx_all.at[k]], hra2.at[b], sems[b])
            pltpu.async_copy(w_hbm.at[didx_all.at[k]], wdr2.at[b], sems[b])
            pltpu.async_copy(hb_hbm.at[didx_all.at[k]], hrb2.at[b], sems[b])
            pltpu.async_copy(w_hbm.at[sidx_all.at[k]], wsr2.at[b], sems[b])
            pltpu.async_copy(exa_hbm.at[pl.ds(base, c2)], exa2.at[b], sems[b])
            pltpu.async_copy(exb_hbm.at[pl.ds(base, c2)], exb2.at[b], sems[b])

        def drain(k, b):
            base = wid * ew + k * c2
            pltpu.make_async_copy(ha_hbm.at[sidx_all.at[k]], hra2.at[b],
                                  sems[b]).wait()
            pltpu.make_async_copy(w_hbm.at[didx_all.at[k]], wdr2.at[b],
                                  sems[b]).wait()
            pltpu.make_async_copy(hb_hbm.at[didx_all.at[k]], hrb2.at[b],
                                  sems[b]).wait()
            pltpu.make_async_copy(w_hbm.at[sidx_all.at[k]], wsr2.at[b],
                                  sems[b]).wait()
            pltpu.make_async_copy(exa_hbm.at[pl.ds(base, c2)], exa2.at[b],
                                  sems[b]).wait()
            pltpu.make_async_copy(exb_hbm.at[pl.ds(base, c2)], exb2.at[b],
                                  sems[b]).wait()

        def reduce_heads(hr, w, m):
            if ch == 16:
                def edge(i, _):
                    wv = [w[i, pl.ds(0, 16)], w[i, pl.ds(16, 16)]]
                    acc = jnp.zeros((16,), jnp.float32)
                    for hh in range(H):
                        acc = acc + (wv[hh // 16][hh % 16]
                                     * hr[i, pl.ds(16 * hh, 16)])
                    m[i, :] = acc
                    return 0
                lax.fori_loop(0, c2, edge, 0)
            else:
                def pair(i2, _):
                    r0 = 2 * i2
                    rowidx = jnp.where(lmask, r0, r0 + 1)
                    wlo = [w[r0, pl.ds(0, 16)], w[r0, pl.ds(16, 16)]]
                    whi = [w[r0 + 1, pl.ds(0, 16)], w[r0 + 1, pl.ds(16, 16)]]
                    acc = jnp.zeros((16,), jnp.float32)
                    for hh in range(H):
                        hv = plsc.load_gather(hr, [rowidx, pat8 + 8 * hh])
                        wv = jnp.where(lmask, wlo[hh // 16][hh % 16],
                                       whi[hh // 16][hh % 16])
                        acc = acc + wv * hv
                    plsc.store_scatter(m, [rowidx, pat8], acc)
                    return 0
                lax.fori_loop(0, c2 // 2, pair, 0)

        def compute(j, b):
            hra = hra2.at[b]
            hrb = hrb2.at[b]
            wsr = wsr2.at[b]
            wdr = wdr2.at[b]
            exa_v = exa2.at[b]
            exb_v = exb2.at[b]

            def wrow(i, _):
                for kk in range(H // 16):
                    o = 16 * kk
                    wa[i, pl.ds(o, 16)] = (exa_v[i, pl.ds(o, 16)]
                                           * wdr[i, pl.ds(o, 16)])
                    wb[i, pl.ds(o, 16)] = (exb_v[i, pl.ds(o, 16)]
                                           * wsr[i, pl.ds(H + o, 16)])
                return 0

            lax.fori_loop(0, c2, wrow, 0)
            reduce_heads(hra, wa, ma)
            reduce_heads(hrb, wb, mb)
            _scatter_add_rows(ma, nsha, didx_all.at[j])
            _scatter_add_rows(mb, nshb, sidx_all.at[j])

        issue(0, 0)

        def pair2(j2, _):
            j = 2 * j2
            issue(j + 1, 1)
            drain(j, 0)
            compute(j, 0)

            @pl.when(j + 2 < nchunk)
            def _():
                issue(j + 2, 0)

            drain(j + 1, 1)
            compute(j + 1, 1)
            return 0

        lax.fori_loop(0, nchunk // 2, pair2, 0)
        plsc.subcore_barrier()
        pltpu.sync_copy(nsha.at[pl.ds(s_ * rpt, rpt)],
                        numa_hbm.at[c, pl.ds(s_ * rpt, rpt)])
        pltpu.sync_copy(nshb.at[pl.ds(s_ * rpt, rpt)],
                        numb_hbm.at[c, pl.ds(s_ * rpt, rpt)])

    return kern(s2d, d2d, h_a, h_b, ex_a, ex_b, w_t, zeros_ch)


def _blockdiag(a):
    ch = a.shape[1]
    eye = jnp.eye(H, dtype=a.dtype)
    return (a[:, :, None] * eye[:, None, :]).reshape(H * ch, H)


def kernel(x, edge_index, gi1_W, gi1_as, gi1_ad, gi1_b, gi2_W, gi2_as,
           gi2_ad, gi2_b, go1_W, go1_as, go1_ad, go1_b, go2_W, go2_as,
           go2_ad, go2_b, ae_e1_W, ae_e1_b, ae_bn1_g, ae_bn1_b, ae_e2_W,
           ae_e2_b, ae_bn2_g, ae_bn2_b, ae_d1_W, ae_d1_b, ae_d2_W, ae_d2_b):
    n, din = x.shape
    e = edge_index.shape[1]
    np_ = -(-(n + 1) // NBLK) * NBLK
    ep = -(-e // (NW * C1)) * (NW * C1)

    xp = jnp.pad(x, ((0, np_ - n), (0, 0)))
    pad_ids = jnp.full((ep - e,), n, dtype=jnp.int32)
    s_ids = jnp.concatenate([edge_index[0], pad_ids])
    d_ids = jnp.concatenate([edge_index[1], pad_ids])
    zeros32 = jnp.zeros((np_, H), jnp.float32)
    zeros16 = jnp.zeros((np_, 16), jnp.float32)
    zeros8 = jnp.zeros((np_, 8), jnp.float32)

    ae = (ae_e1_W.T, ae_e1_b[None, :], ae_bn1_g[None, :], ae_bn1_b[None, :],
          ae_e2_W.T, ae_e2_b[None, :], ae_bn2_g[None, :], ae_bn2_b[None, :],
          ae_d1_W.T, ae_d1_b[None, :], ae_d2_W.T, ae_d2_b[None, :])

    (hgi, esgi, edgi, hgo, esgo, edgo, e1, e2, zre) = _stage1(
        xp, gi1_W.T, _blockdiag(gi1_as), _blockdiag(gi1_ad),
        go1_W.T, _blockdiag(go1_as), _blockdiag(go1_ad), ae)

    u1, v1 = _tables(esgi, edgi, esgo, edgo)

    exgi, exgo, dengi, dengo = _pass1(s_ids, d_ids, u1, v1, zeros32)
    w1 = _ivd(dengi, dengo)
    numgi, numgo = _pass2(s_ids, d_ids, hgi, hgo, exgi, exgo, w1, zeros16, 16)

    (z1, o1, hgi2, esgi2, edgi2, hgo2, esgo2, edgo2) = _layer2(
        numgi, numgo, gi1_b[None, :], go1_b[None, :],
        gi2_W.T, _blockdiag(gi2_as), _blockdiag(gi2_ad),
        go2_W.T, _blockdiag(go2_as), _blockdiag(go2_ad))
    u2, v2 = _tables(esgi2, edgi2, esgo2, edgo2)

    exgi2, exgo2, dengi2, dengo2 = _pass1(s_ids, d_ids, u2, v2, zeros32)
    w2 = _ivd(dengi2, dengo2)
    numgi2, numgo2 = _pass2(s_ids, d_ids, hgi2, hgo2, exgi2, exgo2, w2,
                            zeros8, 8)

    z2, o2 = _final(numgi2, numgo2, gi2_b[None, :], go2_b[None, :])

    x_in = jnp.concatenate([z1[:n], z2[:n]], axis=-1)
    x_out = jnp.concatenate([o1[:n], o2[:n]], axis=-1)
    x_self = jnp.concatenate([e1[:n], e2[:n]], axis=-1)
    z_self_re = zre[:n]
    return (x_in, x_out, x_self, z_self_re)


# speedup vs baseline: 35.8930x; 1.0069x over previous
import functools

import jax
import jax.numpy as jnp
from jax import lax
from jax.experimental import pallas as pl
from jax.experimental.pallas import tpu as pltpu
from jax.experimental.pallas import tpu_sc as plsc

_pallas_call = pl.pallas_call
_sc_kernel = pl.kernel

H = 32
NC = 2
NS = 16
NW = NC * NS
C1 = 128
C2 = 64
NBLK = 1024


def _leaky(t):
    return jnp.maximum(t, 0.2 * t)


def _elu(t):
    return jnp.where(t > 0, t, jnp.exp(jnp.minimum(t, 0.0)) - 1.0)


def _pack_uv(es_a, ed_a, es_b, ed_b):
    z = jnp.zeros_like(es_a)
    m_a = _leaky(jnp.max(es_a, axis=0, keepdims=True) + ed_a)
    m_b = _leaky(jnp.max(es_b, axis=0, keepdims=True) + ed_b)
    u = jnp.concatenate([es_a, ed_b, m_b, z], axis=1)
    v = jnp.concatenate([ed_a, m_a, es_b, z], axis=1)
    return u, v


def _stage1_body(x_ref, wgi, asgi, adgi, wgo, asgo, adgo,
                 ae1w, ae1b, bn1g, bn1b, ae2w, ae2b, bn2g, bn2b,
                 d1w, d1b, d2w, d2b,
                 hgi_o, esgi_o, edgi_o, hgo_o, esgo_o, edgo_o,
                 e1_o, e2_o, zre_o):
    x = x_ref[...]
    hgi = jnp.dot(x, wgi[...], preferred_element_type=jnp.float32)
    hgi_o[...] = hgi
    esgi_o[...] = jnp.dot(hgi, asgi[...], preferred_element_type=jnp.float32)
    edgi_o[...] = jnp.dot(hgi, adgi[...], preferred_element_type=jnp.float32)
    hgo = jnp.dot(x, wgo[...], preferred_element_type=jnp.float32)
    hgo_o[...] = hgo
    esgo_o[...] = jnp.dot(hgo, asgo[...], preferred_element_type=jnp.float32)
    edgo_o[...] = jnp.dot(hgo, adgo[...], preferred_element_type=jnp.float32)
    bn_scale = 1.0 / jnp.sqrt(jnp.float32(1.0 + 1e-5))
    e1 = _elu(jnp.dot(x, ae1w[...], preferred_element_type=jnp.float32)
              + ae1b[...])
    e1 = e1 * bn_scale * bn1g[...] + bn1b[...]
    e1_o[...] = e1
    e2 = _elu(jnp.dot(e1, ae2w[...], preferred_element_type=jnp.float32)
              + ae2b[...])
    e2 = e2 * bn_scale * bn2g[...] + bn2b[...]
    e2_o[...] = e2
    d1 = _elu(jnp.dot(e2, d1w[...], preferred_element_type=jnp.float32)
              + d1b[...])
    zre_o[...] = jax.nn.sigmoid(
        jnp.dot(d1, d2w[...], preferred_element_type=jnp.float32) + d2b[...])


def _stage1(xp, wgi, asgi, adgi, wgo, asgo, adgo, ae):
    np_, din = xp.shape
    hw = wgi.shape[1]
    grid = np_ // NBLK

    def wspec(a):
        return pl.BlockSpec(a.shape, lambda i: tuple(0 for _ in a.shape))

    args = (wgi, asgi, adgi, wgo, asgo, adgo) + ae
    out_shapes = [
        jax.ShapeDtypeStruct((np_, hw), jnp.float32),
        jax.ShapeDtypeStruct((np_, H), jnp.float32),
        jax.ShapeDtypeStruct((np_, H), jnp.float32),
        jax.ShapeDtypeStruct((np_, hw), jnp.float32),
        jax.ShapeDtypeStruct((np_, H), jnp.float32),
        jax.ShapeDtypeStruct((np_, H), jnp.float32),
        jax.ShapeDtypeStruct((np_, 16), jnp.float32),
        jax.ShapeDtypeStruct((np_, 8), jnp.float32),
        jax.ShapeDtypeStruct((np_, din), jnp.float32),
    ]
    return _pallas_call(
        _stage1_body,
        grid=(grid,),
        in_specs=[pl.BlockSpec((NBLK, din), lambda i: (i, 0))]
                 + [wspec(a) for a in args],
        out_specs=[pl.BlockSpec((NBLK, s.shape[1]), lambda i: (i, 0))
                   for s in out_shapes],
        out_shape=out_shapes,
    )(xp, *args)


def _tables_body(esgi, edgi, esgo, edgo, u_o, v_o):
    u, v = _pack_uv(esgi[...], edgi[...], esgo[...], edgo[...])
    u_o[...] = u
    v_o[...] = v


def _tables(esgi, edgi, esgo, edgo):
    np_ = esgi.shape[0]
    out = [jax.ShapeDtypeStruct((np_, 4 * H), jnp.float32)] * 2
    return _pallas_call(_tables_body, out_shape=out)(esgi, edgi, esgo, edgo)


def _extend_body(h_a, h_b, dena, denb, exta_o, extb_o):
    ivda = 1.0 / (dena[0] + dena[1] + 1e-16)
    ivdb = 1.0 / (denb[0] + denb[1] + 1e-16)
    z = jnp.zeros((ivda.shape[0], 128 - H), jnp.float32)
    exta_o[...] = jnp.concatenate([h_a[...], ivdb, z], axis=1)
    extb_o[...] = jnp.concatenate([h_b[...], ivda, z], axis=1)


def _extend(h_a, h_b, dena, denb):
    np_, hw = h_a.shape
    grid = np_ // NBLK

    def rspec(w):
        return pl.BlockSpec((NBLK, w), lambda i: (i, 0))

    den_spec = pl.BlockSpec((NC, NBLK, H), lambda i: (0, i, 0))
    out = [jax.ShapeDtypeStruct((np_, hw + 128), jnp.float32)] * 2
    return _pallas_call(
        _extend_body,
        grid=(grid,),
        in_specs=[rspec(hw), rspec(hw), den_spec, den_spec],
        out_specs=[rspec(hw + 128), rspec(hw + 128)],
        out_shape=out)(h_a, h_b, dena, denb)


def _layer2_body(numgi, numgo, bgi, bgo, wgi2, asgi2, adgi2, wgo2, asgo2,
                 adgo2, z1_o, o1_o, hgi2_o, esgi2_o, edgi2_o, hgo2_o,
                 esgo2_o, edgo2_o):
    z1 = _elu((numgi[0] + numgi[1]) * (1.0 / H) + bgi[...])
    o1 = _elu((numgo[0] + numgo[1]) * (1.0 / H) + bgo[...])
    z1_o[...] = z1
    o1_o[...] = o1
    hgi2 = jnp.dot(z1, wgi2[...], preferred_element_type=jnp.float32)
    hgi2_o[...] = hgi2
    esgi2_o[...] = jnp.dot(hgi2, asgi2[...], preferred_element_type=jnp.float32)
    edgi2_o[...] = jnp.dot(hgi2, adgi2[...], preferred_element_type=jnp.float32)
    hgo2 = jnp.dot(o1, wgo2[...], preferred_element_type=jnp.float32)
    hgo2_o[...] = hgo2
    esgo2_o[...] = jnp.dot(hgo2, asgo2[...], preferred_element_type=jnp.float32)
    edgo2_o[...] = jnp.dot(hgo2, adgo2[...], preferred_element_type=jnp.float32)


def _layer2(numgi, numgo, bgi, bgo, wgi2, asgi2, adgi2, wgo2, asgo2, adgo2):
    np_ = numgi.shape[1]
    hw2 = wgi2.shape[1]
    ch = wgi2.shape[0]
    grid = np_ // NBLK

    def wspec(a):
        return pl.BlockSpec(a.shape, lambda i: tuple(0 for _ in a.shape))

    num_spec = pl.BlockSpec((NC, NBLK, ch), lambda i: (0, i, 0))
    wargs = (bgi, bgo, wgi2, asgi2, adgi2, wgo2, asgo2, adgo2)
    out_shapes = [
        jax.ShapeDtypeStruct((np_, ch), jnp.float32),
        jax.ShapeDtypeStruct((np_, ch), jnp.float32),
        jax.ShapeDtypeStruct((np_, hw2), jnp.float32),
        jax.ShapeDtypeStruct((np_, H), jnp.float32),
        jax.ShapeDtypeStruct((np_, H), jnp.float32),
        jax.ShapeDtypeStruct((np_, hw2), jnp.float32),
        jax.ShapeDtypeStruct((np_, H), jnp.float32),
        jax.ShapeDtypeStruct((np_, H), jnp.float32),
    ]
    return _pallas_call(
        _layer2_body,
        grid=(grid,),
        in_specs=[num_spec, num_spec] + [wspec(a) for a in wargs],
        out_specs=[pl.BlockSpec((NBLK, s.shape[1]), lambda i: (i, 0))
                   for s in out_shapes],
        out_shape=out_shapes,
    )(numgi, numgo, *wargs)


def _final_body(numgi2, numgo2, bgi2, bgo2, z2_o, o2_o):
    z2_o[...] = _elu((numgi2[0] + numgi2[1]) * (1.0 / H) + bgi2[...])
    o2_o[...] = _elu((numgo2[0] + numgo2[1]) * (1.0 / H) + bgo2[...])


def _final(numgi2, numgo2, bgi2, bgo2):
    out = [jax.ShapeDtypeStruct(numgi2.shape[1:], jnp.float32)] * 2
    return _pallas_call(_final_body, out_shape=out)(
        numgi2, numgo2, bgi2, bgo2)


def _scatter_add_rows(src_vmem, shared_ref, idx_ref):
    pltpu.sync_copy(src_vmem, shared_ref.at[idx_ref], add=True)


def _sc_mesh():
    return plsc.VectorSubcoreMesh(core_axis_name="c", subcore_axis_name="s")


def _pass1(s_ids, d_ids, u_t, v_t, zeros32):
    ep = s_ids.shape[0]
    np_ = u_t.shape[0]
    ew = ep // NW
    nchunk = ew // C1
    rpt = np_ // NS
    s2d = s_ids.reshape(ep // C1, C1)
    d2d = d_ids.reshape(ep // C1, C1)

    @functools.partial(
        _sc_kernel, mesh=_sc_mesh(),
        compiler_params=pltpu.CompilerParams(use_tc_tiling_on_sc=False,
                                             needs_layout_passes=False),
        out_type=[jax.ShapeDtypeStruct((ep, H), jnp.float32),
                  jax.ShapeDtypeStruct((ep, H), jnp.float32),
                  jax.ShapeDtypeStruct((NC, np_, H), jnp.float32),
                  jax.ShapeDtypeStruct((NC, np_, H), jnp.float32)],
        scratch_types=[
            pltpu.VMEM((nchunk, C1), jnp.int32),
            pltpu.VMEM((nchunk, C1), jnp.int32),
            pltpu.VMEM((2, C1, 4 * H), jnp.float32),
            pltpu.VMEM((2, C1, 4 * H), jnp.float32),
            pltpu.VMEM((C1, H), jnp.float32), pltpu.VMEM((C1, H), jnp.float32),
            pltpu.VMEM_SHARED((np_, H), jnp.float32),
            pltpu.VMEM_SHARED((np_, H), jnp.float32),
            pltpu.SemaphoreType.DMA, pltpu.SemaphoreType.DMA,
        ],
    )
    def kern(s_hbm, d_hbm, u_hbm, v_hbm, z_hbm,
             exa_hbm, exb_hbm, dena_hbm, denb_hbm,
             sidx_all, didx_all, ur2, vr2, exa_v, exb_v,
             dsha, dshb, sem_a, sem_b):
        c = lax.axis_index("c")
        s_ = lax.axis_index("s")
        wid = c * NS + s_
        pltpu.sync_copy(z_hbm.at[pl.ds(s_ * rpt, rpt)],
                        dsha.at[pl.ds(s_ * rpt, rpt)])
        pltpu.sync_copy(z_hbm.at[pl.ds(s_ * rpt, rpt)],
                        dshb.at[pl.ds(s_ * rpt, rpt)])
        pltpu.sync_copy(s_hbm.at[pl.ds(wid * nchunk, nchunk)], sidx_all)
        pltpu.sync_copy(d_hbm.at[pl.ds(wid * nchunk, nchunk)], didx_all)
        plsc.subcore_barrier()

        sems = (sem_a, sem_b)

        def issue(k, b):
            pltpu.async_copy(u_hbm.at[sidx_all.at[k]], ur2.at[b], sems[b])
            pltpu.async_copy(v_hbm.at[didx_all.at[k]], vr2.at[b], sems[b])

        def drain(k, b):
            pltpu.make_async_copy(u_hbm.at[sidx_all.at[k]], ur2.at[b],
                                  sems[b]).wait()
            pltpu.make_async_copy(v_hbm.at[didx_all.at[k]], vr2.at[b],
                                  sems[b]).wait()

        def compute(j, b):
            ur = ur2.at[b]
            vr = vr2.at[b]

            def row(i, _):
                for kk in range(H // 16):
                    o = 16 * kk
                    ta = ur[i, pl.ds(o, 16)] + vr[i, pl.ds(o, 16)]
                    exa_v[i, pl.ds(o, 16)] = jnp.exp(
                        jnp.maximum(ta, 0.2 * ta) - vr[i, pl.ds(H + o, 16)])
                    tb = vr[i, pl.ds(2 * H + o, 16)] + ur[i, pl.ds(H + o, 16)]
                    exb_v[i, pl.ds(o, 16)] = jnp.exp(
                        jnp.maximum(tb, 0.2 * tb) - ur[i, pl.ds(2 * H + o, 16)])
                return 0

            lax.fori_loop(0, C1, row, 0)
            base = wid * ew + j * C1
            pltpu.sync_copy(exa_v, exa_hbm.at[pl.ds(base, C1)])
            pltpu.sync_copy(exb_v, exb_hbm.at[pl.ds(base, C1)])
            _scatter_add_rows(exa_v, dsha, didx_all.at[j])
            _scatter_add_rows(exb_v, dshb, sidx_all.at[j])

        issue(0, 0)

        def pair(j2, _):
            j = 2 * j2
            issue(j + 1, 1)
            drain(j, 0)
            compute(j, 0)

            @pl.when(j + 2 < nchunk)
            def _():
                issue(j + 2, 0)

            drain(j + 1, 1)
            compute(j + 1, 1)
            return 0

        lax.fori_loop(0, nchunk // 2, pair, 0)
        plsc.subcore_barrier()
        pltpu.sync_copy(dsha.at[pl.ds(s_ * rpt, rpt)],
                        dena_hbm.at[c, pl.ds(s_ * rpt, rpt)])
        pltpu.sync_copy(dshb.at[pl.ds(s_ * rpt, rpt)],
                        denb_hbm.at[c, pl.ds(s_ * rpt, rpt)])

    return kern(s2d, d2d, u_t, v_t, zeros32)


def _pass2(s_ids, d_ids, h_a, h_b, ex_a, ex_b, zeros_ch, ch):
    ep = s_ids.shape[0]
    np_ = h_a.shape[0]
    hw = h_a.shape[1] - 128
    c2 = 32
    ew = ep // NW
    nchunk = ew // c2
    rpt = np_ // NS
    s2d = s_ids.reshape(ep // c2, c2)
    d2d = d_ids.reshape(ep // c2, c2)

    @functools.partial(
        _sc_kernel, mesh=_sc_mesh(),
        compiler_params=pltpu.CompilerParams(use_tc_tiling_on_sc=False,
                                             needs_layout_passes=False),
        out_type=[jax.ShapeDtypeStruct((NC, np_, ch), jnp.float32),
                  jax.ShapeDtypeStruct((NC, np_, ch), jnp.float32)],
        scratch_types=[
            pltpu.VMEM((nchunk, c2), jnp.int32),
            pltpu.VMEM((nchunk, c2), jnp.int32),
            pltpu.VMEM((2, c2, hw + 128), jnp.float32),
            pltpu.VMEM((2, c2, hw + 128), jnp.float32),
            pltpu.VMEM((2, c2, H), jnp.float32),
            pltpu.VMEM((2, c2, H), jnp.float32),
            pltpu.VMEM((c2, H), jnp.float32), pltpu.VMEM((c2, H), jnp.float32),
            pltpu.VMEM((c2, ch), jnp.float32), pltpu.VMEM((c2, ch), jnp.float32),
            pltpu.VMEM_SHARED((np_, ch), jnp.float32),
            pltpu.VMEM_SHARED((np_, ch), jnp.float32),
            pltpu.SemaphoreType.DMA, pltpu.SemaphoreType.DMA,
        ],
    )
    def kern(s_hbm, d_hbm, ha_hbm, hb_hbm, exa_hbm, exb_hbm, z_hbm,
             numa_hbm, numb_hbm,
             sidx_all, didx_all, hra2, hrb2, exa2, exb2, wa, wb,
             ma, mb, nsha, nshb, sem_a, sem_b):
        c = lax.axis_index("c")
        s_ = lax.axis_index("s")
        wid = c * NS + s_
        pltpu.sync_copy(z_hbm.at[pl.ds(s_ * rpt, rpt)],
                        nsha.at[pl.ds(s_ * rpt, rpt)])
        pltpu.sync_copy(z_hbm.at[pl.ds(s_ * rpt, rpt)],
                        nshb.at[pl.ds(s_ * rpt, rpt)])
        pltpu.sync_copy(s_hbm.at[pl.ds(wid * nchunk, nchunk)], sidx_all)
        pltpu.sync_copy(d_hbm.at[pl.ds(wid * nchunk, nchunk)], didx_all)
        plsc.subcore_barrier()

        lmask = lax.iota(jnp.int32, 16) < 8
        pat8 = lax.iota(jnp.int32, 16) & 7
        sems = (sem_a, sem_b)

        def issue(k, b):
            base = wid * ew + k * c2
            pltpu.async_copy(ha_hbm.at[sidx_all.at[k]], hra2.at[b], sems[b])
            pltpu.async_copy(hb_hbm.at[didx_all.at[k]], hrb2.at[b], sems[b])
            pltpu.async_copy(exa_hbm.at[pl.ds(base, c2)], exa2.at[b], sems[b])
            pltpu.async_copy(exb_hbm.at[pl.ds(base, c2)], exb2.at[b], sems[b])

        def drain(k, b):
            base = wid * ew + k * c2
            pltpu.make_async_copy(ha_hbm.at[sidx_all.at[k]], hra2.at[b],
                                  sems[b]).wait()
            pltpu.make_async_copy(hb_hbm.at[didx_all.at[k]], hrb2.at[b],
                                  sems[b]).wait()
            pltpu.make_async_copy(exa_hbm.at[pl.ds(base, c2)], exa2.at[b],
                                  sems[b]).wait()
            pltpu.make_async_copy(exb_hbm.at[pl.ds(base, c2)], exb2.at[b],
                                  sems[b]).wait()

        def reduce_heads(hr, w, m):
            if ch == 16:
                def edge(i, _):
                    wv = [w[i, pl.ds(0, 16)], w[i, pl.ds(16, 16)]]
                    acc = jnp.zeros((16,), jnp.float32)
                    for hh in range(H):
                        acc = acc + (wv[hh // 16][hh % 16]
                                     * hr[i, pl.ds(16 * hh, 16)])
                    m[i, :] = acc
                    return 0
                lax.fori_loop(0, c2, edge, 0)
            else:
                def pair(i2, _):
                    r0 = 2 * i2
                    rowidx = jnp.where(lmask, r0, r0 + 1)
                    wlo = [w[r0, pl.ds(0, 16)], w[r0, pl.ds(16, 16)]]
                    whi = [w[r0 + 1, pl.ds(0, 16)], w[r0 + 1, pl.ds(16, 16)]]
                    acc = jnp.zeros((16,), jnp.float32)
                    for hh in range(H):
                        hv = plsc.load_gather(hr, [rowidx, pat8 + 8 * hh])
                        wv = jnp.where(lmask, wlo[hh // 16][hh % 16],
                                       whi[hh // 16][hh % 16])
                        acc = acc + wv * hv
                    plsc.store_scatter(m, [rowidx, pat8], acc)
                    return 0
                lax.fori_loop(0, c2 // 2, pair, 0)

        def compute(j, b):
            hra = hra2.at[b]
            hrb = hrb2.at[b]
            exa_v = exa2.at[b]
            exb_v = exb2.at[b]

            def wrow(i, _):
                for kk in range(H // 16):
                    o = 16 * kk
                    wa[i, pl.ds(o, 16)] = (exa_v[i, pl.ds(o, 16)]
                                           * hrb[i, pl.ds(hw + o, 16)])
                    wb[i, pl.ds(o, 16)] = (exb_v[i, pl.ds(o, 16)]
                                           * hra[i, pl.ds(hw + o, 16)])
                return 0

            lax.fori_loop(0, c2, wrow, 0)
            reduce_heads(hra, wa, ma)
            reduce_heads(hrb, wb, mb)
            _scatter_add_rows(ma, nsha, didx_all.at[j])
            _scatter_add_rows(mb, nshb, sidx_all.at[j])

        issue(0, 0)

        def pair2(j2, _):
            j = 2 * j2
            issue(j + 1, 1)
            drain(j, 0)
            compute(j, 0)

            @pl.when(j + 2 < nchunk)
            def _():
                issue(j + 2, 0)

            drain(j + 1, 1)
            compute(j + 1, 1)
            return 0

        lax.fori_loop(0, nchunk // 2, pair2, 0)
        plsc.subcore_barrier()
        pltpu.sync_copy(nsha.at[pl.ds(s_ * rpt, rpt)],
                        numa_hbm.at[c, pl.ds(s_ * rpt, rpt)])
        pltpu.sync_copy(nshb.at[pl.ds(s_ * rpt, rpt)],
                        numb_hbm.at[c, pl.ds(s_ * rpt, rpt)])

    return kern(s2d, d2d, h_a, h_b, ex_a, ex_b, zeros_ch)


def _blockdiag(a):
    ch = a.shape[1]
    eye = jnp.eye(H, dtype=a.dtype)
    return (a[:, :, None] * eye[:, None, :]).reshape(H * ch, H)


def kernel(x, edge_index, gi1_W, gi1_as, gi1_ad, gi1_b, gi2_W, gi2_as,
           gi2_ad, gi2_b, go1_W, go1_as, go1_ad, go1_b, go2_W, go2_as,
           go2_ad, go2_b, ae_e1_W, ae_e1_b, ae_bn1_g, ae_bn1_b, ae_e2_W,
           ae_e2_b, ae_bn2_g, ae_bn2_b, ae_d1_W, ae_d1_b, ae_d2_W, ae_d2_b):
    n, din = x.shape
    e = edge_index.shape[1]
    np_ = -(-(n + 1) // NBLK) * NBLK
    ep = -(-e // (NW * C1)) * (NW * C1)

    xp = jnp.pad(x, ((0, np_ - n), (0, 0)))
    pad_ids = jnp.full((ep - e,), n, dtype=jnp.int32)
    s_ids = jnp.concatenate([edge_index[0], pad_ids])
    d_ids = jnp.concatenate([edge_index[1], pad_ids])
    zeros32 = jnp.zeros((np_, H), jnp.float32)
    zeros16 = jnp.zeros((np_, 16), jnp.float32)
    zeros8 = jnp.zeros((np_, 8), jnp.float32)

    ae = (ae_e1_W.T, ae_e1_b[None, :], ae_bn1_g[None, :], ae_bn1_b[None, :],
          ae_e2_W.T, ae_e2_b[None, :], ae_bn2_g[None, :], ae_bn2_b[None, :],
          ae_d1_W.T, ae_d1_b[None, :], ae_d2_W.T, ae_d2_b[None, :])

    (hgi, esgi, edgi, hgo, esgo, edgo, e1, e2, zre) = _stage1(
        xp, gi1_W.T, _blockdiag(gi1_as), _blockdiag(gi1_ad),
        go1_W.T, _blockdiag(go1_as), _blockdiag(go1_ad), ae)

    u1, v1 = _tables(esgi, edgi, esgo, edgo)

    exgi, exgo, dengi, dengo = _pass1(s_ids, d_ids, u1, v1, zeros32)
    exta1, extb1 = _extend(hgi, hgo, dengi, dengo)
    numgi, numgo = _pass2(s_ids, d_ids, exta1, extb1, exgi, exgo, zeros16, 16)

    (z1, o1, hgi2, esgi2, edgi2, hgo2, esgo2, edgo2) = _layer2(
        numgi, numgo, gi1_b[None, :], go1_b[None, :],
        gi2_W.T, _blockdiag(gi2_as), _blockdiag(gi2_ad),
        go2_W.T, _blockdiag(go2_as), _blockdiag(go2_ad))
    u2, v2 = _tables(esgi2, edgi2, esgo2, edgo2)

    exgi2, exgo2, dengi2, dengo2 = _pass1(s_ids, d_ids, u2, v2, zeros32)
    exta2, extb2 = _extend(hgi2, hgo2, dengi2, dengo2)
    numgi2, numgo2 = _pass2(s_ids, d_ids, exta2, extb2, exgi2, exgo2,
                            zeros8, 8)

    z2, o2 = _final(numgi2, numgo2, gi2_b[None, :], go2_b[None, :])

    x_in = jnp.concatenate([z1[:n], z2[:n]], axis=-1)
    x_out = jnp.concatenate([o1[:n], o2[:n]], axis=-1)
    x_self = jnp.concatenate([e1[:n], e2[:n]], axis=-1)
    z_self_re = zre[:n]
    return (x_in, x_out, x_self, z_self_re)
